# baseline (speedup 1.0000x reference)
import functools

import jax
import jax.numpy as jnp
from jax import lax
from jax.experimental import pallas as pl
from jax.experimental.pallas import tpu as pltpu
from jax.experimental.pallas import tpu_sc as plsc

_SCALE = 8.0


@functools.lru_cache(maxsize=None)
def _make_sc_kernel(B, V, D):
    info = plsc.get_sparse_core_info()
    NC, NS, L = info.num_cores, info.num_subcores, info.num_lanes
    NW = NC * NS
    assert B % NW == 0 and D % L == 0
    b_per_w = B // NW
    CH = 128
    assert b_per_w % CH == 0
    n_ch = b_per_w // CH
    mesh = plsc.VectorSubcoreMesh(core_axis_name="c", subcore_axis_name="s")

    @functools.partial(
        pl.kernel,
        mesh=mesh,
        compiler_params=pltpu.CompilerParams(use_tc_tiling_on_sc=False),
        out_type=jax.ShapeDtypeStruct((B, D), jnp.float32),
        scratch_types=[
            pltpu.VMEM((CH,), jnp.int32),
            pltpu.VMEM((CH, D), jnp.float32),
            pltpu.SemaphoreType.DMA,
        ],
    )
    def k(idx_hbm, table_hbm, out_hbm, idx_v, rows_v, sem):
        wid = lax.axis_index("s") * NC + lax.axis_index("c")
        base = wid * b_per_w

        def chunk_body(c, carry):
            off = base + c * CH
            pltpu.sync_copy(idx_hbm.at[pl.ds(off, CH)], idx_v)
            pltpu.async_copy(table_hbm.at[idx_v], rows_v, sem).wait()

            def scale_row(i, carry2):
                for j in range(D // L):
                    sl = pl.ds(j * L, L)
                    rows_v[i, sl] = rows_v[i, sl] * _SCALE
                return carry2

            lax.fori_loop(0, CH, scale_row, 0)
            pltpu.sync_copy(rows_v, out_hbm.at[pl.ds(off, CH)])
            return carry

        lax.fori_loop(0, n_ch, chunk_body, 0)

    return k


def kernel(inputs, table):
    B = inputs.shape[0] * inputs.shape[1]
    D = table.shape[1]
    idx = inputs.reshape(B).astype(jnp.int32)
    out = _make_sc_kernel(B, table.shape[0], D)(idx, table)
    return out.reshape(inputs.shape[0], inputs.shape[1], D)


# speedup vs baseline: 1.2547x; 1.2547x over previous
import functools

import jax
import jax.numpy as jnp
from jax import lax
from jax.experimental import pallas as pl
from jax.experimental.pallas import tpu as pltpu
from jax.experimental.pallas import tpu_sc as plsc

_SCALE = 8.0
_CH = 512
_SUB = _CH // 128


@functools.lru_cache(maxsize=None)
def _make_sc_kernel(B, V, D):
    info = plsc.get_sparse_core_info()
    NC, NS, L = info.num_cores, info.num_subcores, info.num_lanes
    NW = NC * NS
    assert B % (NW * _CH) == 0 and D % L == 0
    b_per_w = B // NW
    n_steps = b_per_w // _CH
    assert n_steps % 2 == 0 and n_steps >= 4
    mesh = plsc.VectorSubcoreMesh(core_axis_name="c", subcore_axis_name="s")

    @functools.partial(
        pl.kernel,
        mesh=mesh,
        compiler_params=pltpu.CompilerParams(use_tc_tiling_on_sc=False),
        out_type=jax.ShapeDtypeStruct((B, D), jnp.float32),
        scratch_types=[
            pltpu.VMEM((2, _SUB, 128), jnp.int32),
            pltpu.VMEM((2, _CH, D), jnp.float32),
            pltpu.SemaphoreType.DMA,
            pltpu.SemaphoreType.DMA,
        ],
    )
    def k(idx_hbm, table_hbm, out_hbm, idx_v, rows_v, gsem, ssem):
        wid = lax.axis_index("s") * NC + lax.axis_index("c")
        base = wid * b_per_w
        base_row = wid * (b_per_w // 128)

        def load_fire(g, b):
            pltpu.sync_copy(idx_hbm.at[pl.ds(base_row + g * _SUB, _SUB)],
                            idx_v.at[b])
            for j in range(_SUB):
                pltpu.async_copy(table_hbm.at[idx_v.at[b, j]],
                                 rows_v.at[b, pl.ds(j * 128, 128)], gsem)

        def wait_gather(b):
            for j in range(_SUB):
                pltpu.make_async_copy(table_hbm.at[idx_v.at[b, j]],
                                      rows_v.at[b, pl.ds(j * 128, 128)],
                                      gsem).wait()

        def fire_store(g, b):
            pltpu.async_copy(rows_v.at[b],
                             out_hbm.at[pl.ds(base + g * _CH, _CH)], ssem)

        def wait_store(g, b):
            pltpu.make_async_copy(rows_v.at[b],
                                  out_hbm.at[pl.ds(base + g * _CH, _CH)],
                                  ssem).wait()

        def scale(b):
            @pl.loop(0, _CH, unroll=4)
            def _scale_row(i):
                for j in range(D // L):
                    sl = pl.ds(j * L, L)
                    rows_v[b, i, sl] = rows_v[b, i, sl] * _SCALE

        load_fire(0, 0)

        @pl.loop(0, n_steps // 2)
        def _pair(g2):
            g = g2 * 2
            wait_gather(0)

            @pl.when(g2 >= 1)
            def _():
                wait_store(g - 1, 1)

            load_fire(g + 1, 1)
            scale(0)
            fire_store(g, 0)
            wait_gather(1)
            wait_store(g, 0)

            @pl.when(g2 + 1 < n_steps // 2)
            def _():
                load_fire(g + 2, 0)

            scale(1)
            fire_store(g + 1, 1)

        wait_store(n_steps - 1, 1)

    return k


def kernel(inputs, table):
    B = inputs.shape[0] * inputs.shape[1]
    D = table.shape[1]
    idx = inputs.reshape(B // 128, 128).astype(jnp.int32)
    out = _make_sc_kernel(B, table.shape[0], D)(idx, table)
    return out.reshape(inputs.shape[0], inputs.shape[1], D)


# speedup vs baseline: 1.2675x; 1.0102x over previous
import functools

import jax
import jax.numpy as jnp
from jax import lax
from jax.experimental import pallas as pl
from jax.experimental.pallas import tpu as pltpu
from jax.experimental.pallas import tpu_sc as plsc

_SCALE = 8.0
_R = 4


@functools.lru_cache(maxsize=None)
def _make_sc_kernel(N, S, V, D):
    info = plsc.get_sparse_core_info()
    NC, NS, L = info.num_cores, info.num_subcores, info.num_lanes
    NW = NC * NS
    assert N % (NW * _R) == 0 and D % L == 0
    rows_per_w = N // NW
    n_steps = rows_per_w // _R
    assert n_steps % 2 == 0 and n_steps >= 4
    pieces = []
    off = 0
    while off < S:
        pieces.append((off, min(128, S - off)))
        off += min(128, S - off)
    mesh = plsc.VectorSubcoreMesh(core_axis_name="c", subcore_axis_name="s")

    @functools.partial(
        pl.kernel,
        mesh=mesh,
        compiler_params=pltpu.CompilerParams(use_tc_tiling_on_sc=False),
        out_type=jax.ShapeDtypeStruct((N, S, D), jnp.float32),
        scratch_types=[
            pltpu.VMEM((2, _R, S), jnp.int32),
            pltpu.VMEM((2, _R, S, D), jnp.float32),
            pltpu.SemaphoreType.DMA,
            pltpu.SemaphoreType.DMA,
        ],
    )
    def k(idx_hbm, table_hbm, out_hbm, idx_v, rows_v, gsem, ssem):
        wid = lax.axis_index("s") * NC + lax.axis_index("c")
        base = wid * rows_per_w

        def load_fire(g, b):
            pltpu.sync_copy(idx_hbm.at[pl.ds(base + g * _R, _R)], idx_v.at[b])
            for r in range(_R):
                for (o, n) in pieces:
                    pltpu.async_copy(
                        table_hbm.at[idx_v.at[b, r, pl.ds(o, n)]],
                        rows_v.at[b, r, pl.ds(o, n)], gsem)

        def wait_gather(b):
            for r in range(_R):
                for (o, n) in pieces:
                    pltpu.make_async_copy(
                        table_hbm.at[idx_v.at[b, r, pl.ds(o, n)]],
                        rows_v.at[b, r, pl.ds(o, n)], gsem).wait()

        def fire_store(g, b):
            pltpu.async_copy(rows_v.at[b],
                             out_hbm.at[pl.ds(base + g * _R, _R)], ssem)

        def wait_store(g, b):
            pltpu.make_async_copy(rows_v.at[b],
                                  out_hbm.at[pl.ds(base + g * _R, _R)],
                                  ssem).wait()

        def scale(b):
            @pl.loop(0, S, unroll=2)
            def _scale_col(i):
                for r in range(_R):
                    for j in range(D // L):
                        sl = pl.ds(j * L, L)
                        rows_v[b, r, i, sl] = rows_v[b, r, i, sl] * _SCALE

        load_fire(0, 0)

        @pl.loop(0, n_steps // 2)
        def _pair(g2):
            g = g2 * 2
            wait_gather(0)

            @pl.when(g2 >= 1)
            def _():
                wait_store(g - 1, 1)

            load_fire(g + 1, 1)
            scale(0)
            fire_store(g, 0)
            wait_gather(1)
            wait_store(g, 0)

            @pl.when(g2 + 1 < n_steps // 2)
            def _():
                load_fire(g + 2, 0)

            scale(1)
            fire_store(g + 1, 1)

        wait_store(n_steps - 1, 1)

    return k


def kernel(inputs, table):
    N, S = inputs.shape
    V, D = table.shape
    out = _make_sc_kernel(N, S, V, D)(inputs.astype(jnp.int32), table)
    return out
